# baseline (speedup 1.0000x reference)
import functools

import jax
import jax.numpy as jnp
from jax import lax
from jax.experimental import pallas as pl
from jax.experimental.pallas import tpu as pltpu
from jax.experimental.pallas import tpu_sc as plsc

_GROUP = 128
_NSUB = 16
_NCORE = 2
_NUM_WORKERS = _NSUB * _NCORE


def kernel(vocab_id_list, table):
    B, L = vocab_id_list.shape
    V, D = table.shape
    N = B * L
    n_groups = N // _GROUP
    g_per_w = n_groups // _NUM_WORKERS

    K = 5
    CH = K * _GROUP
    n_super = g_per_w // K

    idx2d = vocab_id_list.reshape(n_groups, _GROUP)

    mesh = plsc.VectorSubcoreMesh(core_axis_name="c", subcore_axis_name="s")

    @functools.partial(
        pl.kernel,
        out_type=jax.ShapeDtypeStruct((N, D), jnp.float32),
        mesh=mesh,
        compiler_params=pltpu.CompilerParams(use_tc_tiling_on_sc=False),
        scratch_types=[
            pltpu.VMEM((g_per_w, _GROUP), jnp.int32),
            pltpu.VMEM((CH, D), jnp.float32),
            pltpu.VMEM((CH, D), jnp.float32),
            pltpu.VMEM_SHARED((_NSUB * CH, D), jnp.float32),
            pltpu.VMEM_SHARED((_NSUB * CH, D), jnp.float32),
            pltpu.SemaphoreType.DMA,
            pltpu.SemaphoreType.DMA,
            pltpu.SemaphoreType.DMA,
            pltpu.SemaphoreType.DMA,
        ],
    )
    def sc_gather(
        idx_hbm, table_hbm, out_hbm,
        idx_v, t0, t1, bufa, bufb, gsem0, gsem1, osema, osemb,
    ):
        cid = lax.axis_index("c")
        sid = lax.axis_index("s")
        wid = sid * _NCORE + cid
        gbase = wid * g_per_w
        pltpu.sync_copy(idx_hbm.at[pl.ds(gbase, g_per_w)], idx_v)

        def fire(s, tbuf, gsem):
            for k in range(K):
                pltpu.async_copy(
                    table_hbm.at[idx_v.at[s * K + k]],
                    tbuf.at[pl.ds(k * _GROUP, _GROUP)],
                    gsem,
                )

        def drain_g(tbuf, gsem):
            for k in range(K):
                pltpu.make_async_copy(
                    table_hbm.at[pl.ds(0, _GROUP)],
                    tbuf.at[pl.ds(k * _GROUP, _GROUP)],
                    gsem,
                ).wait()

        def push(tbuf, sbuf):
            pltpu.sync_copy(tbuf, sbuf.at[pl.ds(sid * CH, CH)])

        def put(s, sbuf, osem):
            for t in range(_NSUB):
                trows = ((t * _NCORE + cid) * g_per_w + s * K) * _GROUP
                pltpu.async_copy(
                    sbuf.at[pl.ds(t * CH, CH)],
                    out_hbm.at[pl.ds(trows, CH)],
                    osem,
                )

        def drain_o(sbuf, osem):
            for t in range(_NSUB):
                pltpu.make_async_copy(
                    sbuf.at[pl.ds(t * CH, CH)],
                    out_hbm.at[pl.ds(0, CH)],
                    osem,
                ).wait()

        fire(0, t0, gsem0)
        fire(1, t1, gsem1)

        @pl.loop(0, n_super, step=2)
        def _(s):
            drain_g(t0, gsem0)

            @pl.when((sid == 0) & (s >= 2))
            def _():
                drain_o(bufa, osema)

            plsc.subcore_barrier()
            push(t0, bufa)
            plsc.subcore_barrier()

            @pl.when(s + 2 < n_super)
            def _():
                fire(s + 2, t0, gsem0)

            @pl.when(sid == 0)
            def _():
                put(s, bufa, osema)

            drain_g(t1, gsem1)

            @pl.when((sid == 0) & (s >= 2))
            def _():
                drain_o(bufb, osemb)

            plsc.subcore_barrier()
            push(t1, bufb)
            plsc.subcore_barrier()

            @pl.when(s + 3 < n_super)
            def _():
                fire(s + 3, t1, gsem1)

            @pl.when(sid == 0)
            def _():
                put(s + 1, bufb, osemb)

        @pl.when(sid == 0)
        def _():
            drain_o(bufa, osema)
            drain_o(bufb, osemb)

    out = sc_gather(idx2d, table)
    return out.reshape(B, L, D)


# speedup vs baseline: 3.1971x; 3.1971x over previous
import functools

import jax
import jax.numpy as jnp
from jax import lax
from jax.experimental import pallas as pl
from jax.experimental.pallas import tpu as pltpu
from jax.experimental.pallas import tpu_sc as plsc

_GROUP = 128
_NUM_WORKERS = 32


def _sc_gather(idx2d, table, N, D):
    n_groups, _ = idx2d.shape
    g_per_w = n_groups // _NUM_WORKERS

    K = 10
    CH = K * _GROUP
    n_chunks = g_per_w // K

    mesh = plsc.VectorSubcoreMesh(core_axis_name="c", subcore_axis_name="s")

    @functools.partial(
        pl.kernel,
        out_type=jax.ShapeDtypeStruct((N, D), jnp.float32),
        mesh=mesh,
        compiler_params=pltpu.CompilerParams(use_tc_tiling_on_sc=False),
        scratch_types=[
            pltpu.VMEM((g_per_w, _GROUP), jnp.int32),
            pltpu.VMEM((CH, D), jnp.float32),
            pltpu.VMEM((CH, D), jnp.float32),
            pltpu.SemaphoreType.DMA,
        ],
    )
    def sc_gather(idx_hbm, table_hbm, out_hbm, idx_v, buf0, buf1, gsem):
        wid = lax.axis_index("s") * 2 + lax.axis_index("c")
        gbase = wid * g_per_w
        row_base = gbase * _GROUP
        pltpu.sync_copy(idx_hbm.at[pl.ds(gbase, g_per_w)], idx_v)

        def fire(c, buf):
            for k in range(K):
                pltpu.async_copy(
                    table_hbm.at[idx_v.at[c * K + k]],
                    buf.at[pl.ds(k * _GROUP, _GROUP)],
                    gsem,
                )

        def drain(buf):
            for k in range(K):
                pltpu.make_async_copy(
                    table_hbm.at[pl.ds(0, _GROUP)],
                    buf.at[pl.ds(k * _GROUP, _GROUP)],
                    gsem,
                ).wait()

        def put(c, buf):
            pltpu.sync_copy(buf, out_hbm.at[pl.ds(row_base + c * CH, CH)])

        fire(0, buf0)

        @pl.loop(0, n_chunks, step=2)
        def _(c):
            drain(buf0)

            @pl.when(c + 1 < n_chunks)
            def _():
                fire(c + 1, buf1)

            put(c, buf0)

            @pl.when(c + 1 < n_chunks)
            def _():
                drain(buf1)

                @pl.when(c + 2 < n_chunks)
                def _():
                    fire(c + 2, buf0)

                put(c + 1, buf1)

    return sc_gather(idx2d, table)


def _tc_transpose(g, B, L, D):
    BB = 128
    LD = L * D
    n_blocks = B // BB
    g2 = g.reshape(B, LD)

    def body(in_ref, out_ref):
        x = in_ref[...]
        xt = jnp.swapaxes(x, 0, 1)
        out_ref[...] = xt.reshape(L, D // 8, 1, 8, BB)

    return pl.pallas_call(
        body,
        grid=(n_blocks,),
        in_specs=[pl.BlockSpec((BB, LD), lambda i: (i, 0))],
        out_specs=pl.BlockSpec((L, D // 8, 1, 8, BB), lambda i: (0, 0, i, 0, 0)),
        out_shape=jax.ShapeDtypeStruct((L, D // 8, n_blocks, 8, BB), jnp.float32),
        compiler_params=pltpu.CompilerParams(
            dimension_semantics=("parallel",),
        ),
    )(g2)


def kernel(vocab_id_list, table):
    B, L = vocab_id_list.shape
    V, D = table.shape
    N = B * L
    n_groups = N // _GROUP

    idx2d = vocab_id_list.reshape(n_groups, _GROUP)
    g = _sc_gather(idx2d, table, N, D)
    y5 = _tc_transpose(g, B, L, D)
    return y5.transpose(2, 4, 0, 1, 3).reshape(B, L, D)


# speedup vs baseline: 3.2090x; 1.0037x over previous
import functools

import jax
import jax.numpy as jnp
from jax import lax
from jax.experimental import pallas as pl
from jax.experimental.pallas import tpu as pltpu
from jax.experimental.pallas import tpu_sc as plsc

_GROUP = 128
_NUM_WORKERS = 32


def _sc_gather(idx2d, table, N, D):
    n_groups, _ = idx2d.shape
    g_per_w = n_groups // _NUM_WORKERS

    K = 10
    CH = K * _GROUP
    n_chunks = g_per_w // K

    mesh = plsc.VectorSubcoreMesh(core_axis_name="c", subcore_axis_name="s")

    @functools.partial(
        pl.kernel,
        out_type=jax.ShapeDtypeStruct((N, D), jnp.float32),
        mesh=mesh,
        compiler_params=pltpu.CompilerParams(use_tc_tiling_on_sc=False),
        scratch_types=[
            pltpu.VMEM((g_per_w, _GROUP), jnp.int32),
            pltpu.VMEM((CH, D), jnp.float32),
            pltpu.VMEM((CH, D), jnp.float32),
            pltpu.SemaphoreType.DMA,
        ],
    )
    def sc_gather(idx_hbm, table_hbm, out_hbm, idx_v, buf0, buf1, gsem):
        wid = lax.axis_index("s") * 2 + lax.axis_index("c")
        gbase = wid * g_per_w
        row_base = gbase * _GROUP
        pltpu.sync_copy(idx_hbm.at[pl.ds(gbase, g_per_w)], idx_v)

        def fire(c, buf):
            for k in range(K):
                pltpu.async_copy(
                    table_hbm.at[idx_v.at[c * K + k]],
                    buf.at[pl.ds(k * _GROUP, _GROUP)],
                    gsem,
                )

        def drain(buf):
            for k in range(K):
                pltpu.make_async_copy(
                    table_hbm.at[pl.ds(0, _GROUP)],
                    buf.at[pl.ds(k * _GROUP, _GROUP)],
                    gsem,
                ).wait()

        def put(c, buf):
            pltpu.sync_copy(buf, out_hbm.at[pl.ds(row_base + c * CH, CH)])

        fire(0, buf0)

        @pl.loop(0, n_chunks, step=2)
        def _(c):
            drain(buf0)

            @pl.when(c + 1 < n_chunks)
            def _():
                fire(c + 1, buf1)

            put(c, buf0)

            @pl.when(c + 1 < n_chunks)
            def _():
                drain(buf1)

                @pl.when(c + 2 < n_chunks)
                def _():
                    fire(c + 2, buf0)

                put(c + 1, buf1)

    return sc_gather(idx2d, table)


def _tc_transpose(g, B, L, D):
    BB = 128
    LD = L * D
    n_blocks = B // BB
    g2 = g.reshape(B, LD)

    def body(in_hbm, out_ref, vbuf2, sem2):
        i = pl.program_id(0)
        sel = jax.lax.rem(i, 2)

        @pl.when(i == 0)
        def _():
            pltpu.make_async_copy(
                in_hbm.at[pl.ds(0, BB), :], vbuf2.at[0], sem2.at[0]
            ).start()

        @pl.when(i + 1 < n_blocks)
        def _():
            pltpu.make_async_copy(
                in_hbm.at[pl.ds((i + 1) * BB, BB), :],
                vbuf2.at[1 - sel],
                sem2.at[1 - sel],
            ).start()

        pltpu.make_async_copy(
            in_hbm.at[pl.ds(i * BB, BB), :], vbuf2.at[sel], sem2.at[sel]
        ).wait()
        x = vbuf2[sel]
        xt = jnp.swapaxes(x, 0, 1)
        out_ref[...] = xt.reshape(L, D // 8, 1, 8, BB)

    return pl.pallas_call(
        body,
        grid=(n_blocks,),
        in_specs=[pl.BlockSpec(memory_space=pl.ANY)],
        out_specs=pl.BlockSpec((L, D // 8, 1, 8, BB), lambda i: (0, 0, i, 0, 0)),
        out_shape=jax.ShapeDtypeStruct((L, D // 8, n_blocks, 8, BB), jnp.float32),
        scratch_shapes=[
            pltpu.VMEM((2, BB, LD), jnp.float32),
            pltpu.SemaphoreType.DMA((2,)),
        ],
        compiler_params=pltpu.CompilerParams(
            dimension_semantics=("arbitrary",),
        ),
    )(g2)


def kernel(vocab_id_list, table):
    B, L = vocab_id_list.shape
    V, D = table.shape
    N = B * L
    n_groups = N // _GROUP

    idx2d = vocab_id_list.reshape(n_groups, _GROUP)
    g = _sc_gather(idx2d, table, N, D)
    y5 = _tc_transpose(g, B, L, D)
    return y5.transpose(2, 4, 0, 1, 3).reshape(B, L, D)
